# Initial kernel scaffold; baseline (speedup 1.0000x reference)
import functools

import jax
import jax.numpy as jnp
from jax.experimental import pallas as pl

_VOCAB = 113
_VPAD = 128
_D = 512
_L = 200


def _pos_encoding(max_length, d_model):
    even_i = jnp.arange(0, d_model, 2).astype(jnp.float32)
    denominator = jnp.power(jnp.float32(10000.0), even_i / d_model)
    position = jnp.arange(max_length, dtype=jnp.float32).reshape(max_length, 1)
    even_pe = jnp.sin(position / denominator)
    odd_pe = jnp.cos(position / denominator)
    return jnp.stack([even_pe, odd_pe], axis=2).reshape(max_length, d_model)


def _tc_body(x_ref, table_ref, pe_ref, out_ref):
    b = x_ref.shape[0]
    idx = x_ref[...].reshape(b * _L, 1)
    onehot = (idx == jax.lax.broadcasted_iota(jnp.int32, (b * _L, _VPAD), 1))
    emb = jnp.dot(onehot.astype(jnp.bfloat16), table_ref[...],
                  preferred_element_type=jnp.float32)
    out_ref[...] = emb.reshape(b, _L, _D) + pe_ref[...][None, :, :]


@functools.partial(jax.jit, static_argnames=("block_b",))
def _tc_lookup(x, table_pad_bf16, pe, block_b=8):
    batch = x.shape[0]
    grid = (batch // block_b,)
    return pl.pallas_call(
        _tc_body,
        grid=grid,
        in_specs=[
            pl.BlockSpec((block_b, _L), lambda i: (i, 0)),
            pl.BlockSpec((_VPAD, _D), lambda i: (0, 0)),
            pl.BlockSpec((_L, _D), lambda i: (0, 0)),
        ],
        out_specs=pl.BlockSpec((block_b, _L, _D), lambda i: (i, 0, 0)),
        out_shape=jax.ShapeDtypeStruct((batch, _L, _D), jnp.float32),
    )(x, table_pad_bf16, pe)


def kernel(x, table):
    pe = _pos_encoding(_L, _D)
    table_pad = jnp.zeros((_VPAD, _D), jnp.bfloat16).at[:_VOCAB].set(
        table.astype(jnp.bfloat16))
    return _tc_lookup(x.astype(jnp.int32), table_pad, pe)


# speedup vs baseline: 4.2324x; 4.2324x over previous
import functools

import jax
import jax.numpy as jnp
from jax.experimental import pallas as pl

_VOCAB = 113
_VPAD = 128
_D = 512
_L = 200


def _pos_encoding(max_length, d_model):
    even_i = jnp.arange(0, d_model, 2).astype(jnp.float32)
    denominator = jnp.power(jnp.float32(10000.0), even_i / d_model)
    position = jnp.arange(max_length, dtype=jnp.float32).reshape(max_length, 1)
    even_pe = jnp.sin(position / denominator)
    odd_pe = jnp.cos(position / denominator)
    return jnp.stack([even_pe, odd_pe], axis=2).reshape(max_length, d_model)


def _tc_body(x_ref, table_ref, pe_ref, out_ref):
    n = x_ref.shape[0]
    onehot = (x_ref[...] ==
              jax.lax.broadcasted_iota(jnp.int32, (n, _VPAD), 1))
    emb = jnp.dot(onehot.astype(jnp.bfloat16), table_ref[...],
                  preferred_element_type=jnp.float32)
    out_ref[...] = emb + pe_ref[...]


@functools.partial(jax.jit, static_argnames=("block_b",))
def _tc_lookup(x2, table_pad_bf16, pe_tiled, block_b=8):
    n_tokens = x2.shape[0]
    rows = block_b * _L
    grid = (n_tokens // rows,)
    return pl.pallas_call(
        _tc_body,
        grid=grid,
        in_specs=[
            pl.BlockSpec((rows, 1), lambda i: (i, 0)),
            pl.BlockSpec((_VPAD, _D), lambda i: (0, 0)),
            pl.BlockSpec((rows, _D), lambda i: (0, 0)),
        ],
        out_specs=pl.BlockSpec((rows, _D), lambda i: (i, 0)),
        out_shape=jax.ShapeDtypeStruct((n_tokens, _D), jnp.float32),
    )(x2, table_pad_bf16, pe_tiled)


def kernel(x, table, block_b=8):
    batch, length = x.shape
    pe = _pos_encoding(_L, _D)
    pe_tiled = jnp.tile(pe, (block_b, 1))
    table_pad = jnp.zeros((_VPAD, _D), jnp.bfloat16).at[:_VOCAB].set(
        table.astype(jnp.bfloat16))
    x2 = x.astype(jnp.int32).reshape(batch * length, 1)
    out = _tc_lookup(x2, table_pad, pe_tiled, block_b=block_b)
    return out.reshape(batch, length, _D)
